# Initial kernel scaffold; baseline (speedup 1.0000x reference)
import jax
import jax.numpy as jnp
from jax.experimental import pallas as pl


def kernel(local_features, global_features, Wg1, bg1, Wg2, bg2, Wfine, bfine, Wc, a_src, a_dst, Wc2, Wg, a_src2, a_dst2, Wout, bout):
    raise NotImplementedError("write your pallas kernel here")


# speedup vs baseline: 1.8634x; 1.8634x over previous
import functools

import jax
import jax.numpy as jnp
import numpy as np
from jax.experimental import pallas as pl
from jax.experimental.pallas import tpu as pltpu

_B, _N, _CD = 64, 38, 768
_C = 512
_HEADS, _ATT = 8, 64
_HD = _HEADS * _ATT
_GPROJ, _DG, _LLM = 2048, 1024, 4096
_R = 6
_REGION = tuple(int(i * _R // _N) for i in range(_N))
_STARTS = (0, 7, 13, 19, 26, 32)
_ENDS = (7, 13, 19, 26, 32, 38)
_CNTS = (7, 6, 6, 7, 6, 6)


def _leaky(x):
    return jnp.maximum(x, 0.0) + 0.2 * jnp.minimum(x, 0.0)


def _elu(x):
    return jnp.where(x > 0.0, x, jnp.exp(jnp.minimum(x, 0.0)) - 1.0)


def _head_onehot(rows, lanes, scale_col=None):
    di = jax.lax.broadcasted_iota(jnp.int32, (rows, lanes), 0) // _ATT
    li = jax.lax.broadcasted_iota(jnp.int32, (rows, lanes), 1)
    oh = jnp.where(di == li, 1.0, 0.0)
    if scale_col is not None:
        oh = oh * scale_col
    return oh


def _pool_body(gf_ref, out_ref):
    q = jax.lax.broadcasted_iota(jnp.int32, (256, 32), 0)
    p = jax.lax.broadcasted_iota(jnp.int32, (256, 32), 1)
    h, w, d = q // 32, (q // 4) % 8, q % 4
    i1, j1, k1 = p // 8, (p // 2) % 4, p % 2
    P = jnp.where((h // 2 == i1) & (w // 2 == j1) & (d // 2 == k1), 0.125, 0.0)
    out_ref[...] = jnp.dot(gf_ref[...], P, preferred_element_type=jnp.float32)


def _gmlp_body(x_ref, Wg1_ref, bg1_ref, Wg2_ref, bg2_ref, Wgp_ref, a2d_ref,
               e2d_ref, acc_ref):
    k = pl.program_id(0)

    @pl.when(k == 0)
    def _():
        acc_ref[...] = jnp.zeros_like(acc_ref)

    acc_ref[...] += jnp.dot(x_ref[...], Wg1_ref[...],
                            preferred_element_type=jnp.float32)

    @pl.when(k == pl.num_programs(0) - 1)
    def _():
        t = jnp.maximum(acc_ref[...] + bg1_ref[...], 0.0)
        g = jnp.dot(t, Wg2_ref[...], preferred_element_type=jnp.float32) \
            + bg2_ref[...]
        A2 = _head_onehot(_HD, 16, a2d_ref[...])
        Wgd = jnp.dot(Wgp_ref[...], A2, preferred_element_type=jnp.float32)
        e2d_ref[...] = jnp.dot(g, Wgd, preferred_element_type=jnp.float32)


def _fine_body(x_ref, Wf_ref, bf_ref, Wc_ref, asrc_ref, adst_ref,
               hf_ref, E_ref):
    h = jnp.dot(x_ref[...], Wf_ref[...],
                preferred_element_type=jnp.float32) + bf_ref[...]
    hf_ref[...] = h
    Hsrc = _head_onehot(_HD, 16, asrc_ref[...])
    Wed = jnp.dot(Wc_ref[...], _head_onehot(_HD, 16, adst_ref[...]),
                  preferred_element_type=jnp.float32)
    Wlog = jnp.concatenate([Hsrc, Wed], axis=1)
    E_ref[...] = jnp.dot(h, Wlog, preferred_element_type=jnp.float32)


def _gat_body(hf_ref, E_ref, e2d_ref, Wc2_ref, a2s_ref, co_ref, go_ref, nb):
    li = jax.lax.broadcasted_iota(jnp.int32, (16, _HD), 0)
    di = jax.lax.broadcasted_iota(jnp.int32, (16, _HD), 1) // _ATT
    HT = jnp.where(li == di, 1.0, 0.0)
    Hs2 = _head_onehot(_HD, 16, a2s_ref[...])
    go_rows = []
    for i in range(nb):
        hfi = hf_ref[i]
        Ei = E_ref[i]
        co_rows = []
        for r in range(_R):
            s, e, c = _STARTS[r], _ENDS[r], _CNTS[r]
            ed = jnp.mean(Ei[s:e, 16:32], axis=0, keepdims=True)
            er = _leaky(Ei[s:e, 0:16] + ed)
            m = jnp.max(er, axis=0, keepdims=True)
            ee = jnp.exp(er - m)
            al = ee / jnp.sum(ee, axis=0, keepdims=True)
            aexp = jnp.dot(al, HT, preferred_element_type=jnp.float32)
            cr = jnp.sum(aexp * hfi[s:e], axis=0, keepdims=True)
            co_rows.append(_elu(cr))
        co_i = jnp.concatenate(co_rows, axis=0)
        co_ref[i] = co_i
        hs2 = jnp.dot(co_i, Wc2_ref[...], preferred_element_type=jnp.float32)
        e2s = jnp.dot(hs2, Hs2, preferred_element_type=jnp.float32)
        e2 = _leaky(e2s + e2d_ref[i:i + 1, :])
        m2 = jnp.max(e2, axis=0, keepdims=True)
        ee2 = jnp.exp(e2 - m2)
        al2 = ee2 / jnp.sum(ee2, axis=0, keepdims=True)
        a2e = jnp.dot(al2, HT, preferred_element_type=jnp.float32)
        go_rows.append(_elu(jnp.sum(a2e * hs2, axis=0, keepdims=True)))
    go_ref[...] = jnp.concatenate(go_rows, axis=0)


def _out_body(hf_ref, co_ref, go_ref, Wa_ref, Wb_ref, Wc3_ref, bout_ref,
              out_ref, nb):
    bf = jnp.bfloat16
    hf = hf_ref[...].reshape(nb * _N, _HD).astype(bf)
    co = co_ref[...].reshape(nb * _R, _HD).astype(bf)
    go = go_ref[...].astype(bf)
    acc = jnp.dot(hf, Wa_ref[...].astype(bf), preferred_element_type=jnp.float32)
    cco = jnp.dot(co, Wb_ref[...].astype(bf), preferred_element_type=jnp.float32)
    gco = jnp.dot(go, Wc3_ref[...].astype(bf), preferred_element_type=jnp.float32)
    n = jax.lax.broadcasted_iota(jnp.int32, (nb * _N, nb * _R), 0)
    k = jax.lax.broadcasted_iota(jnp.int32, (nb * _N, nb * _R), 1)
    ridx = (n // _N) * _R + ((n % _N) * _R) // _N
    OH = jnp.where(k == ridx, 1.0, 0.0)
    n2 = jax.lax.broadcasted_iota(jnp.int32, (nb * _N, nb), 0)
    b2 = jax.lax.broadcasted_iota(jnp.int32, (nb * _N, nb), 1)
    OHg = jnp.where(b2 == n2 // _N, 1.0, 0.0)
    acc = acc + jnp.dot(OH, cco, preferred_element_type=jnp.float32)
    acc = acc + jnp.dot(OHg, gco, preferred_element_type=jnp.float32)
    cb = out_ref.shape[2]
    out_ref[...] = (acc + bout_ref[...]).reshape(nb, _N, cb)


def kernel(local_features, global_features, Wg1, bg1, Wg2, bg2, Wfine, bfine,
           Wc, a_src, a_dst, Wc2, Wg, a_src2, a_dst2, Wout, bout):
    f32 = jnp.float32
    gf = global_features.reshape(_B * _C, 256)
    pool_rows = 2048
    pooled = pl.pallas_call(
        _pool_body,
        grid=(_B * _C // pool_rows,),
        in_specs=[pl.BlockSpec((pool_rows, 256), lambda i: (i, 0))],
        out_specs=pl.BlockSpec((pool_rows, 32), lambda i: (i, 0)),
        out_shape=jax.ShapeDtypeStruct((_B * _C, 32), f32),
    )(gf)
    x = pooled.reshape(_B, _C * 32)

    kb = 2048
    nk = _C * 32 // kb
    e2d = pl.pallas_call(
        _gmlp_body,
        grid=(nk,),
        in_specs=[
            pl.BlockSpec((_B, kb), lambda k: (0, k)),
            pl.BlockSpec((kb, _GPROJ), lambda k: (k, 0)),
            pl.BlockSpec((1, _GPROJ), lambda k: (0, 0)),
            pl.BlockSpec((_GPROJ, _DG), lambda k: (0, 0)),
            pl.BlockSpec((1, _DG), lambda k: (0, 0)),
            pl.BlockSpec((_DG, _HD), lambda k: (0, 0)),
            pl.BlockSpec((_HD, 1), lambda k: (0, 0)),
        ],
        out_specs=pl.BlockSpec((_B, 16), lambda k: (0, 0)),
        out_shape=jax.ShapeDtypeStruct((_B, 16), f32),
        scratch_shapes=[pltpu.VMEM((_B, _GPROJ), f32)],
    )(x, Wg1, bg1.reshape(1, -1), Wg2, bg2.reshape(1, -1), Wg,
      a_dst2.reshape(_HD, 1))

    xf = local_features.reshape(_B * _N, _CD)
    rb = 128
    nr = (_B * _N) // rb
    h_fine, E = pl.pallas_call(
        _fine_body,
        grid=(nr,),
        in_specs=[
            pl.BlockSpec((rb, _CD), lambda i: (i, 0)),
            pl.BlockSpec((_CD, _HD), lambda i: (0, 0)),
            pl.BlockSpec((1, _HD), lambda i: (0, 0)),
            pl.BlockSpec((_HD, _HD), lambda i: (0, 0)),
            pl.BlockSpec((_HD, 1), lambda i: (0, 0)),
            pl.BlockSpec((_HD, 1), lambda i: (0, 0)),
        ],
        out_specs=[
            pl.BlockSpec((rb, _HD), lambda i: (i, 0)),
            pl.BlockSpec((rb, 32), lambda i: (i, 0)),
        ],
        out_shape=[
            jax.ShapeDtypeStruct((_B * _N, _HD), f32),
            jax.ShapeDtypeStruct((_B * _N, 32), f32),
        ],
    )(xf, Wfine, bfine.reshape(1, -1), Wc, a_src.reshape(_HD, 1),
      a_dst.reshape(_HD, 1))

    hf3 = h_fine.reshape(_B, _N, _HD)
    E3 = E.reshape(_B, _N, 32)
    nb = 8
    co3, go = pl.pallas_call(
        functools.partial(_gat_body, nb=nb),
        grid=(_B // nb,),
        in_specs=[
            pl.BlockSpec((nb, _N, _HD), lambda i: (i, 0, 0)),
            pl.BlockSpec((nb, _N, 32), lambda i: (i, 0, 0)),
            pl.BlockSpec((nb, 16), lambda i: (i, 0)),
            pl.BlockSpec((_HD, _HD), lambda i: (0, 0)),
            pl.BlockSpec((_HD, 1), lambda i: (0, 0)),
        ],
        out_specs=[
            pl.BlockSpec((nb, _R, _HD), lambda i: (i, 0, 0)),
            pl.BlockSpec((nb, _HD), lambda i: (i, 0)),
        ],
        out_shape=[
            jax.ShapeDtypeStruct((_B, _R, _HD), f32),
            jax.ShapeDtypeStruct((_B, _HD), f32),
        ],
    )(hf3, E3, e2d, Wc2, a_src2.reshape(_HD, 1))

    nb2 = 8
    cb = 2048
    out = pl.pallas_call(
        functools.partial(_out_body, nb=nb2),
        grid=(_LLM // cb, _B // nb2),
        in_specs=[
            pl.BlockSpec((nb2, _N, _HD), lambda j, i: (i, 0, 0)),
            pl.BlockSpec((nb2, _R, _HD), lambda j, i: (i, 0, 0)),
            pl.BlockSpec((nb2, _HD), lambda j, i: (i, 0)),
            pl.BlockSpec((_HD, cb), lambda j, i: (0, j)),
            pl.BlockSpec((_HD, cb), lambda j, i: (1, j)),
            pl.BlockSpec((_HD, cb), lambda j, i: (2, j)),
            pl.BlockSpec((1, cb), lambda j, i: (0, j)),
        ],
        out_specs=pl.BlockSpec((nb2, _N, cb), lambda j, i: (i, 0, j)),
        out_shape=jax.ShapeDtypeStruct((_B, _N, _LLM), f32),
    )(hf3, co3, go, Wout, Wout, Wout, bout.reshape(1, -1))
    return out
